# baseline (speedup 1.0000x reference)
import functools

import jax
import jax.numpy as jnp
from jax import lax
from jax.experimental import pallas as pl
from jax.experimental.pallas import tpu as pltpu
from jax.experimental.pallas import tpu_sc as plsc

N = 10000
E = 320000
H = 128

NC = 2
NS = 16
NW = NC * NS
EK = 64
EC = 162
EPT = EK * EC
EPAD = NW * EPT
DEEP = 3

DK = 128
DC = EPT // DK

NPAD = 10112
STRIPE = NPAD // NS

RB = 400
GRID = N // RB

_BN_S = 0.9999950000374997

_mesh = plsc.VectorSubcoreMesh(core_axis_name="c", subcore_axis_name="s")


@functools.partial(
    pl.kernel,
    out_type=jax.ShapeDtypeStruct((NC, NPAD, H), jnp.float32),
    mesh=_mesh,
    scratch_types=[
        pltpu.VMEM((DC, DK), jnp.int32),
        pltpu.VMEM((DK, H), jnp.float32),
        pltpu.VMEM_SHARED((NPAD, H), jnp.float32),
    ],
)
def _deg_kernel(dstr_hbm, zbig_hbm, e0_hbm, out_hbm, dst_v, ones_v, hist_sh):
    c = lax.axis_index("c")
    s = lax.axis_index("s")
    wid = c * NS + s
    pltpu.sync_copy(dstr_hbm.at[wid], dst_v)
    pltpu.sync_copy(e0_hbm, ones_v)
    pltpu.sync_copy(zbig_hbm.at[pl.ds(s * STRIPE, STRIPE)],
                    hist_sh.at[pl.ds(s * STRIPE, STRIPE)])
    plsc.subcore_barrier()

    @pl.loop(0, DC)
    def _(j):
        pltpu.sync_copy(ones_v, hist_sh.at[dst_v.at[j]], add=True)

    plsc.subcore_barrier()
    pltpu.sync_copy(hist_sh.at[pl.ds(s * STRIPE, STRIPE)],
                    out_hbm.at[c, pl.ds(s * STRIPE, STRIPE)])


@functools.partial(
    pl.kernel,
    out_type=jax.ShapeDtypeStruct((NC, NPAD, H), jnp.float32),
    mesh=_mesh,
    scratch_types=[
        pltpu.VMEM((EPT,), jnp.int32),
        pltpu.VMEM((EPT,), jnp.int32),
        pltpu.VMEM((EK, H), jnp.float32),
        pltpu.VMEM((EK, H), jnp.float32),
        pltpu.VMEM((EK, H), jnp.float32),
        pltpu.VMEM_SHARED((NPAD, H), jnp.float32),
        pltpu.SemaphoreType.DMA,
        pltpu.SemaphoreType.DMA,
        pltpu.SemaphoreType.DMA,
    ],
)
def _edge_kernel(y_hbm, srcf_hbm, dstf_hbm, zbig_hbm, out_hbm,
                 src_v, dst_v, b0, b1, b2, acc_sh, s0, s1, s2):
    c = lax.axis_index("c")
    s = lax.axis_index("s")
    wid = c * NS + s
    pltpu.sync_copy(srcf_hbm.at[wid], src_v)
    pltpu.sync_copy(dstf_hbm.at[wid], dst_v)
    pltpu.sync_copy(zbig_hbm.at[pl.ds(s * STRIPE, STRIPE)],
                    acc_sh.at[pl.ds(s * STRIPE, STRIPE)])
    plsc.subcore_barrier()

    bufs = (b0, b1, b2)
    sems = (s0, s1, s2)

    def _src(j):
        return src_v.at[pl.ds(j * EK, EK)]

    def _dst(j):
        return dst_v.at[pl.ds(j * EK, EK)]

    for d in range(DEEP):
        pltpu.make_async_copy(y_hbm.at[_src(d)], bufs[d], sems[d]).start()

    @pl.loop(0, EC // DEEP)
    def _(jj):
        base = jj * DEEP
        for d in range(DEEP):
            j = base + d
            pltpu.make_async_copy(y_hbm.at[_src(j)], bufs[d], sems[d]).wait()
            pltpu.sync_copy(bufs[d], acc_sh.at[_dst(j)], add=True)

            @pl.when(j + DEEP < EC)
            def _():
                pltpu.make_async_copy(y_hbm.at[_src(j + DEEP)], bufs[d],
                                      sems[d]).start()

    plsc.subcore_barrier()
    pltpu.sync_copy(acc_sh.at[pl.ds(s * STRIPE, STRIPE)],
                    out_hbm.at[c, pl.ds(s * STRIPE, STRIPE)])


def _mlp_body(x_ref, w0_ref, b0_ref, g0_ref, be0_ref,
              w1_ref, b1_ref, g1_ref, be1_ref, wg0_ref, xw0_ref):
    h = jnp.dot(x_ref[...], w0_ref[...], preferred_element_type=jnp.float32)
    h = jnp.maximum(h + b0_ref[...], 0.0) * g0_ref[...] + be0_ref[...]
    h = jnp.dot(h, w1_ref[...], preferred_element_type=jnp.float32)
    h = jnp.maximum(h + b1_ref[...], 0.0) * g1_ref[...] + be1_ref[...]
    xw0_ref[...] = jnp.dot(h, wg0_ref[...], preferred_element_type=jnp.float32)


def _scale_body(xw_ref, degp_ref, y0_ref, dinvb_ref):
    cnt = degp_ref[0, :, 0:1] + degp_ref[1, :, 0:1]
    dinv = lax.rsqrt(cnt + 1.0)
    dinvb = jnp.broadcast_to(dinv, (RB, H))
    y0_ref[...] = xw_ref[...] * dinvb
    dinvb_ref[...] = dinvb


def _mid_body(p_ref, xw_ref, dinvb_ref, wg1_ref, bg_ref, gg_ref, beg_ref,
              xw1_ref, y1_ref):
    dinvb = dinvb_ref[...]
    t = (p_ref[0] + p_ref[1]) * dinvb + xw_ref[...] * dinvb * dinvb + bg_ref[...]
    out0 = jnp.maximum(t, 0.0) * gg_ref[...] + beg_ref[...]
    xw1 = jnp.dot(out0, wg1_ref[...], preferred_element_type=jnp.float32)
    xw1_ref[...] = xw1
    y1_ref[...] = xw1 * dinvb


def _fin_body(q_ref, xw_ref, dinvb_ref, bg_ref, gg_ref, beg_ref, o_ref):
    dinvb = dinvb_ref[...]
    t = (q_ref[0] + q_ref[1]) * dinvb + xw_ref[...] * dinvb * dinvb + bg_ref[...]
    o_ref[...] = jnp.maximum(t, 0.0) * gg_ref[...] + beg_ref[...]


_row_spec = pl.BlockSpec((RB, H), lambda i: (i, 0))
_w_spec = pl.BlockSpec((H, H), lambda i: (0, 0))
_v_spec = pl.BlockSpec((1, H), lambda i: (0, 0))
_acc_spec = pl.BlockSpec((NC, RB, H), lambda i: (0, i, 0))

_mlp_call = pl.pallas_call(
    _mlp_body,
    grid=(GRID,),
    in_specs=[_row_spec, _w_spec, _v_spec, _v_spec, _v_spec,
              _w_spec, _v_spec, _v_spec, _v_spec, _w_spec],
    out_specs=_row_spec,
    out_shape=jax.ShapeDtypeStruct((N, H), jnp.float32),
)

_scale_call = pl.pallas_call(
    _scale_body,
    grid=(GRID,),
    in_specs=[_row_spec, _acc_spec],
    out_specs=[_row_spec, _row_spec],
    out_shape=[jax.ShapeDtypeStruct((N, H), jnp.float32)] * 2,
)

_mid_call = pl.pallas_call(
    _mid_body,
    grid=(GRID,),
    in_specs=[_acc_spec, _row_spec, _row_spec, _w_spec,
              _v_spec, _v_spec, _v_spec],
    out_specs=[_row_spec, _row_spec],
    out_shape=[jax.ShapeDtypeStruct((N, H), jnp.float32)] * 2,
)

_fin_call = pl.pallas_call(
    _fin_body,
    grid=(GRID,),
    in_specs=[_acc_spec, _row_spec, _row_spec, _v_spec, _v_spec, _v_spec],
    out_specs=_row_spec,
    out_shape=jax.ShapeDtypeStruct((N, H), jnp.float32),
)


def kernel(x, edge_index, w_mlp0, b_mlp0, gamma_mlp0, beta_mlp0,
           w_mlp1, b_mlp1, gamma_mlp1, beta_mlp1,
           w_gcn0, b_gcn0, gamma_gcn0, beta_gcn0,
           w_gcn1, b_gcn1, gamma_gcn1, beta_gcn1):
    src = edge_index[0]
    dst = edge_index[1]
    pad = EPAD - E
    srcf = jnp.concatenate(
        [src, jnp.zeros((pad,), jnp.int32)]).reshape(NW, EPT)
    dstp = jnp.concatenate([dst, jnp.full((pad,), N, jnp.int32)])
    dstf = dstp.reshape(NW, EPT)
    dstr = dstp.reshape(NW, DC, DK)
    zbig = jnp.zeros((NPAD, H), jnp.float32)
    e0rows = jnp.tile(
        (jnp.arange(H) == 0).astype(jnp.float32)[None, :], (DK, 1))

    degp = _deg_kernel(dstr, zbig, e0rows)

    row = lambda v: v.reshape(1, H)
    xw0 = _mlp_call(
        x, w_mlp0, row(b_mlp0), row(gamma_mlp0 * _BN_S), row(beta_mlp0),
        w_mlp1, row(b_mlp1), row(gamma_mlp1 * _BN_S), row(beta_mlp1),
        w_gcn0)
    y0, dinvb = _scale_call(xw0, degp)

    p = _edge_kernel(y0, srcf, dstf, zbig)
    xw1, y1 = _mid_call(p, xw0, dinvb, w_gcn1,
                        row(b_gcn0), row(gamma_gcn0 * _BN_S), row(beta_gcn0))

    q = _edge_kernel(y1, srcf, dstf, zbig)
    out = _fin_call(q, xw1, dinvb,
                    row(b_gcn1), row(gamma_gcn1 * _BN_S), row(beta_gcn1))
    return out


# speedup vs baseline: 1.3621x; 1.3621x over previous
import functools

import jax
import jax.numpy as jnp
from jax import lax
from jax.experimental import pallas as pl
from jax.experimental.pallas import tpu as pltpu
from jax.experimental.pallas import tpu_sc as plsc

N = 10000
E = 320000
H = 128

NC = 2
NS = 16
NW = NC * NS
EK = 80
EC = 128
EPT = EK * EC
EPAD = NW * EPT
DEEP = 2

DK = 80
DC = EPT // DK

NPAD = 10112
STRIPE = NPAD // NS

RB = 400
GRID = N // RB

_BN_S = 0.9999950000374997

_mesh = plsc.VectorSubcoreMesh(core_axis_name="c", subcore_axis_name="s")


@functools.partial(
    pl.kernel,
    out_type=jax.ShapeDtypeStruct((NC, NPAD, H), jnp.float32),
    mesh=_mesh,
    scratch_types=[
        pltpu.VMEM((DC, DK), jnp.int32),
        pltpu.VMEM((DK, H), jnp.float32),
        pltpu.VMEM_SHARED((NPAD, H), jnp.float32),
    ],
)
def _deg_kernel(dstr_hbm, zbig_hbm, e0_hbm, out_hbm, dst_v, ones_v, hist_sh):
    c = lax.axis_index("c")
    s = lax.axis_index("s")
    wid = c * NS + s
    pltpu.sync_copy(dstr_hbm.at[wid], dst_v)
    pltpu.sync_copy(e0_hbm, ones_v)
    pltpu.sync_copy(zbig_hbm.at[pl.ds(s * STRIPE, STRIPE)],
                    hist_sh.at[pl.ds(s * STRIPE, STRIPE)])
    plsc.subcore_barrier()

    @pl.loop(0, DC)
    def _(j):
        pltpu.sync_copy(ones_v, hist_sh.at[dst_v.at[j]], add=True)

    plsc.subcore_barrier()
    pltpu.sync_copy(hist_sh.at[pl.ds(s * STRIPE, STRIPE)],
                    out_hbm.at[c, pl.ds(s * STRIPE, STRIPE)])


@functools.partial(
    pl.kernel,
    out_type=jax.ShapeDtypeStruct((NC, NPAD, H), jnp.float32),
    mesh=_mesh,
    scratch_types=[
        pltpu.VMEM((EPT,), jnp.int32),
        pltpu.VMEM((EC, EK), jnp.int32),
        pltpu.VMEM((EK, H), jnp.float32),
        pltpu.VMEM((EK, H), jnp.float32),
        pltpu.VMEM_SHARED((NPAD, H), jnp.float32),
        pltpu.SemaphoreType.DMA,
        pltpu.SemaphoreType.DMA,
    ],
)
def _edge_kernel(y_hbm, srcf_hbm, dstf_hbm, zbig_hbm, out_hbm,
                 src_v, dst_v, b0, b1, acc_sh, s0, s1):
    c = lax.axis_index("c")
    s = lax.axis_index("s")
    wid = c * NS + s
    pltpu.sync_copy(srcf_hbm.at[wid], src_v)
    pltpu.sync_copy(dstf_hbm.at[wid], dst_v)
    pltpu.sync_copy(zbig_hbm.at[pl.ds(s * STRIPE, STRIPE)],
                    acc_sh.at[pl.ds(s * STRIPE, STRIPE)])
    plsc.subcore_barrier()

    bufs = (b0, b1)
    sems = (s0, s1)

    def _src(j):
        return src_v.at[pl.ds(j * EK, EK)]

    def _dst(j):
        return dst_v.at[j]

    for d in range(DEEP):
        pltpu.make_async_copy(y_hbm.at[_src(d)], bufs[d], sems[d]).start()

    @pl.loop(0, EC // DEEP)
    def _(jj):
        base = jj * DEEP
        for d in range(DEEP):
            j = base + d
            pltpu.make_async_copy(y_hbm.at[_src(j)], bufs[d], sems[d]).wait()
            pltpu.sync_copy(bufs[d], acc_sh.at[_dst(j)], add=True)

            @pl.when(j + DEEP < EC)
            def _():
                pltpu.make_async_copy(y_hbm.at[_src(j + DEEP)], bufs[d],
                                      sems[d]).start()

    plsc.subcore_barrier()
    pltpu.sync_copy(acc_sh.at[pl.ds(s * STRIPE, STRIPE)],
                    out_hbm.at[c, pl.ds(s * STRIPE, STRIPE)])


def _mlp_body(x_ref, w0_ref, b0_ref, g0_ref, be0_ref,
              w1_ref, b1_ref, g1_ref, be1_ref, wg0_ref, xw0_ref):
    h = jnp.dot(x_ref[...], w0_ref[...], preferred_element_type=jnp.float32)
    h = jnp.maximum(h + b0_ref[...], 0.0) * g0_ref[...] + be0_ref[...]
    h = jnp.dot(h, w1_ref[...], preferred_element_type=jnp.float32)
    h = jnp.maximum(h + b1_ref[...], 0.0) * g1_ref[...] + be1_ref[...]
    xw0_ref[...] = jnp.dot(h, wg0_ref[...], preferred_element_type=jnp.float32)


def _scale_body(xw_ref, degp_ref, y0_ref, dinvb_ref):
    cnt = degp_ref[0, :, 0:1] + degp_ref[1, :, 0:1]
    dinv = lax.rsqrt(cnt + 1.0)
    dinvb = jnp.broadcast_to(dinv, (RB, H))
    y0_ref[...] = xw_ref[...] * dinvb
    dinvb_ref[...] = dinvb


def _mid_body(p_ref, xw_ref, dinvb_ref, wg1_ref, bg_ref, gg_ref, beg_ref,
              xw1_ref, y1_ref):
    dinvb = dinvb_ref[...]
    t = (p_ref[0] + p_ref[1]) * dinvb + xw_ref[...] * dinvb * dinvb + bg_ref[...]
    out0 = jnp.maximum(t, 0.0) * gg_ref[...] + beg_ref[...]
    xw1 = jnp.dot(out0, wg1_ref[...], preferred_element_type=jnp.float32)
    xw1_ref[...] = xw1
    y1_ref[...] = xw1 * dinvb


def _fin_body(q_ref, xw_ref, dinvb_ref, bg_ref, gg_ref, beg_ref, o_ref):
    dinvb = dinvb_ref[...]
    t = (q_ref[0] + q_ref[1]) * dinvb + xw_ref[...] * dinvb * dinvb + bg_ref[...]
    o_ref[...] = jnp.maximum(t, 0.0) * gg_ref[...] + beg_ref[...]


_row_spec = pl.BlockSpec((RB, H), lambda i: (i, 0))
_w_spec = pl.BlockSpec((H, H), lambda i: (0, 0))
_v_spec = pl.BlockSpec((1, H), lambda i: (0, 0))
_acc_spec = pl.BlockSpec((NC, RB, H), lambda i: (0, i, 0))

_mlp_call = pl.pallas_call(
    _mlp_body,
    grid=(GRID,),
    in_specs=[_row_spec, _w_spec, _v_spec, _v_spec, _v_spec,
              _w_spec, _v_spec, _v_spec, _v_spec, _w_spec],
    out_specs=_row_spec,
    out_shape=jax.ShapeDtypeStruct((N, H), jnp.float32),
)

_scale_call = pl.pallas_call(
    _scale_body,
    grid=(GRID,),
    in_specs=[_row_spec, _acc_spec],
    out_specs=[_row_spec, _row_spec],
    out_shape=[jax.ShapeDtypeStruct((N, H), jnp.float32)] * 2,
)

_mid_call = pl.pallas_call(
    _mid_body,
    grid=(GRID,),
    in_specs=[_acc_spec, _row_spec, _row_spec, _w_spec,
              _v_spec, _v_spec, _v_spec],
    out_specs=[_row_spec, _row_spec],
    out_shape=[jax.ShapeDtypeStruct((N, H), jnp.float32)] * 2,
)

_fin_call = pl.pallas_call(
    _fin_body,
    grid=(GRID,),
    in_specs=[_acc_spec, _row_spec, _row_spec, _v_spec, _v_spec, _v_spec],
    out_specs=_row_spec,
    out_shape=jax.ShapeDtypeStruct((N, H), jnp.float32),
)


def kernel(x, edge_index, w_mlp0, b_mlp0, gamma_mlp0, beta_mlp0,
           w_mlp1, b_mlp1, gamma_mlp1, beta_mlp1,
           w_gcn0, b_gcn0, gamma_gcn0, beta_gcn0,
           w_gcn1, b_gcn1, gamma_gcn1, beta_gcn1):
    src = edge_index[0]
    dst = edge_index[1]
    pad = EPAD - E
    srcf = jnp.concatenate(
        [src, jnp.zeros((pad,), jnp.int32)]).reshape(NW, EPT)
    dstp = jnp.concatenate([dst, jnp.full((pad,), N, jnp.int32)])
    dstf = dstp.reshape(NW, EC, EK)
    dstr = dstp.reshape(NW, DC, DK)
    zbig = jnp.zeros((NPAD, H), jnp.float32)
    e0rows = jnp.tile(
        (jnp.arange(H) == 0).astype(jnp.float32)[None, :], (DK, 1))

    degp = _deg_kernel(dstr, zbig, e0rows)

    row = lambda v: v.reshape(1, H)
    xw0 = _mlp_call(
        x, w_mlp0, row(b_mlp0), row(gamma_mlp0 * _BN_S), row(beta_mlp0),
        w_mlp1, row(b_mlp1), row(gamma_mlp1 * _BN_S), row(beta_mlp1),
        w_gcn0)
    y0, dinvb = _scale_call(xw0, degp)

    p = _edge_kernel(y0, srcf, dstf, zbig)
    xw1, y1 = _mid_call(p, xw0, dinvb, w_gcn1,
                        row(b_gcn0), row(gamma_gcn0 * _BN_S), row(beta_gcn0))

    q = _edge_kernel(y1, srcf, dstf, zbig)
    out = _fin_call(q, xw1, dinvb,
                    row(b_gcn1), row(gamma_gcn1 * _BN_S), row(beta_gcn1))
    return out


# speedup vs baseline: 1.3797x; 1.0130x over previous
import functools

import jax
import jax.numpy as jnp
from jax import lax
from jax.experimental import pallas as pl
from jax.experimental.pallas import tpu as pltpu
from jax.experimental.pallas import tpu_sc as plsc

N = 10000
E = 320000
H = 128

NC = 2
NS = 16
NW = NC * NS
EK = 80
EC = 128
EPT = EK * EC
EPAD = NW * EPT
DEEP = 2

DK = 80
DC = EPT // DK

NPAD = 10112
STRIPE = NPAD // NS

RB = 400
GRID = N // RB

_BN_S = 0.9999950000374997

_mesh = plsc.VectorSubcoreMesh(core_axis_name="c", subcore_axis_name="s")


@functools.partial(
    pl.kernel,
    out_type=jax.ShapeDtypeStruct((NC, NPAD, H), jnp.float32),
    mesh=_mesh,
    scratch_types=[
        pltpu.VMEM((DC, DK), jnp.int32),
        pltpu.VMEM((DK, H), jnp.float32),
        pltpu.VMEM_SHARED((NPAD, H), jnp.float32),
    ],
)
def _deg_kernel(dstr_hbm, zbig_hbm, e0_hbm, out_hbm, dst_v, ones_v, hist_sh):
    c = lax.axis_index("c")
    s = lax.axis_index("s")
    wid = c * NS + s
    pltpu.sync_copy(dstr_hbm.at[wid], dst_v)
    pltpu.sync_copy(e0_hbm, ones_v)
    pltpu.sync_copy(zbig_hbm.at[pl.ds(s * STRIPE, STRIPE)],
                    hist_sh.at[pl.ds(s * STRIPE, STRIPE)])
    plsc.subcore_barrier()

    @pl.loop(0, DC)
    def _(j):
        pltpu.sync_copy(ones_v, hist_sh.at[dst_v.at[j]], add=True)

    plsc.subcore_barrier()
    pltpu.sync_copy(hist_sh.at[pl.ds(s * STRIPE, STRIPE)],
                    out_hbm.at[c, pl.ds(s * STRIPE, STRIPE)])


@functools.partial(
    pl.kernel,
    out_type=jax.ShapeDtypeStruct((NC, NPAD, H), jnp.float32),
    mesh=_mesh,
    scratch_types=[
        pltpu.VMEM((EPT,), jnp.int32),
        pltpu.VMEM((EC, EK), jnp.int32),
        pltpu.VMEM((EK, H), jnp.float32),
        pltpu.VMEM((EK, H), jnp.float32),
        pltpu.VMEM_SHARED((NPAD, H), jnp.float32),
        pltpu.SemaphoreType.DMA,
        pltpu.SemaphoreType.DMA,
    ],
)
def _edge_kernel(y_hbm, srcf_hbm, dstf_hbm, zbig_hbm, out_hbm,
                 src_v, dst_v, b0, b1, acc_sh, s0, s1):
    c = lax.axis_index("c")
    s = lax.axis_index("s")
    wid = c * NS + s
    pltpu.sync_copy(srcf_hbm.at[wid], src_v)
    pltpu.sync_copy(dstf_hbm.at[wid], dst_v)
    pltpu.sync_copy(zbig_hbm.at[pl.ds(s * STRIPE, STRIPE)],
                    acc_sh.at[pl.ds(s * STRIPE, STRIPE)])
    plsc.subcore_barrier()

    bufs = (b0, b1)
    sems = (s0, s1)

    def _src(j):
        return src_v.at[pl.ds(j * EK, EK)]

    def _dst(j):
        return dst_v.at[j]

    for d in range(DEEP):
        pltpu.make_async_copy(y_hbm.at[_src(d)], bufs[d], sems[d]).start()

    @pl.loop(0, EC // DEEP)
    def _(jj):
        base = jj * DEEP
        for d in range(DEEP):
            j = base + d
            pltpu.make_async_copy(y_hbm.at[_src(j)], bufs[d], sems[d]).wait()
            pltpu.sync_copy(bufs[d], acc_sh.at[_dst(j)], add=True)

            @pl.when(j + DEEP < EC)
            def _():
                pltpu.make_async_copy(y_hbm.at[_src(j + DEEP)], bufs[d],
                                      sems[d]).start()

    plsc.subcore_barrier()
    pltpu.sync_copy(acc_sh.at[pl.ds(s * STRIPE, STRIPE)],
                    out_hbm.at[c, pl.ds(s * STRIPE, STRIPE)])


def _mlp_body(x_ref, w0_ref, b0_ref, g0_ref, be0_ref,
              w1_ref, b1_ref, g1_ref, be1_ref, wg0_ref, degp_ref,
              xw0_ref, y0_ref, dinvb_ref):
    h = jnp.dot(x_ref[...], w0_ref[...], preferred_element_type=jnp.float32)
    h = jnp.maximum(h + b0_ref[...], 0.0) * g0_ref[...] + be0_ref[...]
    h = jnp.dot(h, w1_ref[...], preferred_element_type=jnp.float32)
    h = jnp.maximum(h + b1_ref[...], 0.0) * g1_ref[...] + be1_ref[...]
    xw0 = jnp.dot(h, wg0_ref[...], preferred_element_type=jnp.float32)
    cnt = degp_ref[0, :, 0:1] + degp_ref[1, :, 0:1]
    dinv = lax.rsqrt(cnt + 1.0)
    dinvb = jnp.broadcast_to(dinv, (RB, H))
    xw0_ref[...] = xw0
    y0_ref[...] = xw0 * dinvb
    dinvb_ref[...] = dinvb


def _mid_body(p_ref, xw_ref, dinvb_ref, wg1_ref, bg_ref, gg_ref, beg_ref,
              xw1_ref, y1_ref):
    dinvb = dinvb_ref[...]
    t = (p_ref[0] + p_ref[1]) * dinvb + xw_ref[...] * dinvb * dinvb + bg_ref[...]
    out0 = jnp.maximum(t, 0.0) * gg_ref[...] + beg_ref[...]
    xw1 = jnp.dot(out0, wg1_ref[...], preferred_element_type=jnp.float32)
    xw1_ref[...] = xw1
    y1_ref[...] = xw1 * dinvb


def _fin_body(q_ref, xw_ref, dinvb_ref, bg_ref, gg_ref, beg_ref, o_ref):
    dinvb = dinvb_ref[...]
    t = (q_ref[0] + q_ref[1]) * dinvb + xw_ref[...] * dinvb * dinvb + bg_ref[...]
    o_ref[...] = jnp.maximum(t, 0.0) * gg_ref[...] + beg_ref[...]


_row_spec = pl.BlockSpec((RB, H), lambda i: (i, 0))
_w_spec = pl.BlockSpec((H, H), lambda i: (0, 0))
_v_spec = pl.BlockSpec((1, H), lambda i: (0, 0))
_acc_spec = pl.BlockSpec((NC, RB, H), lambda i: (0, i, 0))

_mlp_call = pl.pallas_call(
    _mlp_body,
    grid=(GRID,),
    in_specs=[_row_spec, _w_spec, _v_spec, _v_spec, _v_spec,
              _w_spec, _v_spec, _v_spec, _v_spec, _w_spec, _acc_spec],
    out_specs=[_row_spec, _row_spec, _row_spec],
    out_shape=[jax.ShapeDtypeStruct((N, H), jnp.float32)] * 3,
)

_mid_call = pl.pallas_call(
    _mid_body,
    grid=(GRID,),
    in_specs=[_acc_spec, _row_spec, _row_spec, _w_spec,
              _v_spec, _v_spec, _v_spec],
    out_specs=[_row_spec, _row_spec],
    out_shape=[jax.ShapeDtypeStruct((N, H), jnp.float32)] * 2,
)

_fin_call = pl.pallas_call(
    _fin_body,
    grid=(GRID,),
    in_specs=[_acc_spec, _row_spec, _row_spec, _v_spec, _v_spec, _v_spec],
    out_specs=_row_spec,
    out_shape=jax.ShapeDtypeStruct((N, H), jnp.float32),
)


def kernel(x, edge_index, w_mlp0, b_mlp0, gamma_mlp0, beta_mlp0,
           w_mlp1, b_mlp1, gamma_mlp1, beta_mlp1,
           w_gcn0, b_gcn0, gamma_gcn0, beta_gcn0,
           w_gcn1, b_gcn1, gamma_gcn1, beta_gcn1):
    src = edge_index[0]
    dst = edge_index[1]
    pad = EPAD - E
    srcf = jnp.concatenate(
        [src, jnp.zeros((pad,), jnp.int32)]).reshape(NW, EPT)
    dstp = jnp.concatenate([dst, jnp.full((pad,), N, jnp.int32)])
    dstf = dstp.reshape(NW, EC, EK)
    dstr = dstp.reshape(NW, DC, DK)
    zbig = jnp.zeros((NPAD, H), jnp.float32)
    e0rows = jnp.tile(
        (jnp.arange(H) == 0).astype(jnp.float32)[None, :], (DK, 1))

    degp = _deg_kernel(dstr, zbig, e0rows)

    row = lambda v: v.reshape(1, H)
    xw0, y0, dinvb = _mlp_call(
        x, w_mlp0, row(b_mlp0), row(gamma_mlp0 * _BN_S), row(beta_mlp0),
        w_mlp1, row(b_mlp1), row(gamma_mlp1 * _BN_S), row(beta_mlp1),
        w_gcn0, degp)

    p = _edge_kernel(y0, srcf, dstf, zbig)
    xw1, y1 = _mid_call(p, xw0, dinvb, w_gcn1,
                        row(b_gcn0), row(gamma_gcn0 * _BN_S), row(beta_gcn0))

    q = _edge_kernel(y1, srcf, dstf, zbig)
    out = _fin_call(q, xw1, dinvb,
                    row(b_gcn1), row(gamma_gcn1 * _BN_S), row(beta_gcn1))
    return out
